# baseline (speedup 1.0000x reference)
import functools

import jax
import jax.numpy as jnp
from jax import lax
from jax.experimental import pallas as pl
from jax.experimental.pallas import tpu as pltpu
from jax.experimental.pallas import tpu_sc as plsc

N = 10000
F = 128
H = 256
C = 40

HALF = 5000
HPAD = 5120
NP = 2 * HPAD
ROWS_PER_TILE = NP // 32

E_TILE = 10240
EP = 32 * E_TILE
STAGE_E = 2048
N_STAGES = E_TILE // STAGE_E
CHUNK = 64
N_CHUNKS = STAGE_E // CHUNK


def _sc_agg_body(h_hbm, src_hbm, dst_hbm, z_hbm, out_hbm,
                 dst_v, src_v, psrc_b, row_b, rows0, rows1, gsem, asem):
    cid = lax.axis_index("c")
    sid = lax.axis_index("s")
    wid = cid * 16 + sid
    tile_base = wid * E_TILE

    part = out_hbm.at[wid]
    for k in range(NP // ROWS_PER_TILE):
        pltpu.sync_copy(z_hbm, part.at[pl.ds(k * ROWS_PER_TILE,
                                             ROWS_PER_TILE)])

    rows_b = [rows0, rows1]

    @pl.loop(0, N_STAGES)
    def _stage(st):
        eb = tile_base + st * STAGE_E
        pltpu.sync_copy(dst_hbm.at[pl.ds(eb, STAGE_E)], dst_v)
        pltpu.sync_copy(src_hbm.at[pl.ds(eb, STAGE_E)], src_v)

        per_row = CHUNK // 16
        for i in range(STAGE_E // 16):
            sv = src_v[pl.ds(i * 16, 16)]
            dv = dst_v[pl.ds(i * 16, 16)]
            psrc = sv + jnp.where(sv >= HALF, 120, 0).astype(jnp.int32)
            psrc_b[i // per_row, pl.ds((i % per_row) * 16, 16)] = psrc
            prow = jnp.where(dv < HALF, dv,
                             jnp.where(dv < N, dv + 120, HALF + (dv & 63)))
            row_b[i // per_row, pl.ds((i % per_row) * 16, 16)] = prow

        ah = [None]
        for c in range(N_CHUNKS):
            b = c & 1
            pltpu.async_copy(h_hbm.at[psrc_b.at[c]], rows_b[b], gsem).wait()
            if ah[0] is not None:
                ah[0].wait()
            ah[0] = pltpu.async_copy(
                rows_b[b], part.at[row_b.at[c]], asem, add=True)
        ah[0].wait()


@functools.cache
def _get_sc_agg():
    mesh = plsc.VectorSubcoreMesh(core_axis_name="c", subcore_axis_name="s")
    return pl.kernel(
        _sc_agg_body,
        mesh=mesh,
        out_type=jax.ShapeDtypeStruct((32, NP, H), jnp.float32),
        scratch_types=[
            pltpu.VMEM((STAGE_E,), jnp.int32),
            pltpu.VMEM((STAGE_E,), jnp.int32),
            pltpu.VMEM((N_CHUNKS, CHUNK), jnp.int32),
            pltpu.VMEM((N_CHUNKS, CHUNK), jnp.int32),
            pltpu.VMEM((CHUNK, H), jnp.float32),
            pltpu.VMEM((CHUNK, H), jnp.float32),
            pltpu.SemaphoreType.DMA,
            pltpu.SemaphoreType.DMA,
        ],
    )


_BR = 512
_GRID = NP // _BR


def _mm_body(x_ref, w_ref, b_ref, o_ref):
    o_ref[...] = (jnp.dot(x_ref[...], w_ref[...],
                          preferred_element_type=jnp.float32) + b_ref[...])


def _ln_relu_mm_body(ap_ref, g_ref, bt_ref, w_ref, b_ref, o_ref):
    a = jnp.sum(ap_ref[...], axis=0)
    mu = jnp.mean(a, axis=1, keepdims=True)
    var = jnp.mean((a - mu) ** 2, axis=1, keepdims=True)
    hn = (a - mu) * lax.rsqrt(var + 1e-5) * g_ref[...] + bt_ref[...]
    h = jnp.maximum(hn, 0.0)
    o_ref[...] = (jnp.dot(h, w_ref[...],
                          preferred_element_type=jnp.float32) + b_ref[...])


def _ln_relu_out_body(ap_ref, g_ref, bt_ref, w_ref, b_ref, o_ref):
    a = jnp.sum(ap_ref[...], axis=0)
    mu = jnp.mean(a, axis=1, keepdims=True)
    var = jnp.mean((a - mu) ** 2, axis=1, keepdims=True)
    hn = (a - mu) * lax.rsqrt(var + 1e-5) * g_ref[...] + bt_ref[...]
    h = jnp.maximum(hn, 0.0)
    logits = (jnp.dot(h, w_ref[...],
                      preferred_element_type=jnp.float32) + b_ref[...])
    col = lax.broadcasted_iota(jnp.int32, logits.shape, 1)
    neg = jnp.where(col < C, logits, -jnp.inf)
    m = jnp.max(neg, axis=1, keepdims=True)
    lse = m + jnp.log(jnp.sum(jnp.exp(neg - m), axis=1, keepdims=True))
    o_ref[...] = logits - lse


def _row_block(width):
    return pl.BlockSpec((_BR, width), lambda i: (i, 0))


def _const_block(shape):
    return pl.BlockSpec(shape, lambda i: (0, 0))


def _mm(x, w, b):
    fin = x.shape[1]
    return pl.pallas_call(
        _mm_body,
        grid=(_GRID,),
        in_specs=[_row_block(fin), _const_block((fin, H)), _const_block((1, H))],
        out_specs=_row_block(H),
        out_shape=jax.ShapeDtypeStruct((NP, H), jnp.float32),
    )(x, w, b.reshape(1, H))


def _part_block():
    return pl.BlockSpec((32, _BR, H), lambda i: (0, i, 0))


def _ln_relu_mm(ap, g, bt, w, b):
    return pl.pallas_call(
        _ln_relu_mm_body,
        grid=(_GRID,),
        in_specs=[_part_block(), _const_block((1, H)),
                  _const_block((1, H)), _const_block((H, H)),
                  _const_block((1, H))],
        out_specs=_row_block(H),
        out_shape=jax.ShapeDtypeStruct((NP, H), jnp.float32),
    )(ap, g.reshape(1, H), bt.reshape(1, H), w, b.reshape(1, H))


def _ln_relu_out(ap, g, bt, wp, bp):
    return pl.pallas_call(
        _ln_relu_out_body,
        grid=(_GRID,),
        in_specs=[_part_block(), _const_block((1, H)),
                  _const_block((1, H)), _const_block((H, 128)),
                  _const_block((1, 128))],
        out_specs=_row_block(128),
        out_shape=jax.ShapeDtypeStruct((NP, 128), jnp.float32),
    )(ap, g.reshape(1, H), bt.reshape(1, H), wp, bp)


def kernel(x, edge_index, W0, b0, g0, bt0, W1, b1, g1, bt1,
           W2, b2, g2, bt2, Wout, bout):
    xp = jnp.zeros((NP, F), jnp.float32)
    xp = xp.at[0:HALF].set(x[0:HALF]).at[HPAD:HPAD + HALF].set(x[HALF:N])

    dst = edge_index[0]
    src = edge_index[1]
    npad = EP - dst.shape[0]
    dstp = jnp.concatenate([dst, jnp.full((npad,), 1 << 28, jnp.int32)])
    srcp = jnp.concatenate([src, jnp.zeros((npad,), jnp.int32)])
    z = jnp.zeros((ROWS_PER_TILE, H), jnp.float32)

    woutp = jnp.zeros((H, 128), jnp.float32).at[:, :C].set(Wout)
    boutp = jnp.zeros((1, 128), jnp.float32).at[0, :C].set(bout)

    sc_agg = _get_sc_agg()
    h = _mm(xp, W0, b0)
    ap = sc_agg(h, srcp, dstp, z)
    h = _ln_relu_mm(ap, g0, bt0, W1, b1)
    ap = sc_agg(h, srcp, dstp, z)
    h = _ln_relu_mm(ap, g1, bt1, W2, b2)
    ap = sc_agg(h, srcp, dstp, z)
    outp = _ln_relu_out(ap, g2, bt2, woutp, boutp)

    return jnp.concatenate([outp[0:HALF], outp[HPAD:HPAD + HALF]])[:, :C]


# speedup vs baseline: 10.8957x; 10.8957x over previous
import functools

import jax
import jax.numpy as jnp
from jax import lax
from jax.experimental import pallas as pl
from jax.experimental.pallas import tpu as pltpu
from jax.experimental.pallas import tpu_sc as plsc

N = 10000
F = 128
H = 256
C = 40

HALF = 5000
HPAD = 5120
NP = 2 * HPAD
ROWS_PER_TILE = NP // 32

E_TILE = 10240
EP = 32 * E_TILE
STAGE_E = 2048
N_STAGES = E_TILE // STAGE_E
CHUNK = 64
N_CHUNKS = STAGE_E // CHUNK
ZR = 128


def _sc_agg_body(h_hbm, src_hbm, dst_hbm, out_hbm,
                 dst_v, src_v, psrc_b, row_b, rows0, rows1, zbuf,
                 gsem, asem, zsem0, zsem1, zsem2, zsem3):
    cid = lax.axis_index("c")
    sid = lax.axis_index("s")
    wid = cid * 16 + sid
    tile_base = wid * E_TILE

    part = out_hbm.at[wid]

    @pl.loop(0, ZR)
    def _(r):
        @pl.loop(0, H // 16)
        def _(l):
            zbuf[r, pl.ds(l * 16, 16)] = jnp.zeros((16,), jnp.float32)

    zsems = [zsem0, zsem1, zsem2, zsem3]
    zhs = [None, None, None, None]
    for k in range(NP // ZR):
        b = k % 4
        if zhs[b] is not None:
            zhs[b].wait()
        zhs[b] = pltpu.async_copy(zbuf, part.at[pl.ds(k * ZR, ZR)], zsems[b])
    for zh in zhs:
        zh.wait()

    rows_b = [rows0, rows1]

    @pl.loop(0, N_STAGES)
    def _stage(st):
        eb = tile_base + st * STAGE_E
        pltpu.sync_copy(dst_hbm.at[pl.ds(eb, STAGE_E)], dst_v)
        pltpu.sync_copy(src_hbm.at[pl.ds(eb, STAGE_E)], src_v)

        per_row = CHUNK // 16
        for i in range(STAGE_E // 16):
            sv = src_v[pl.ds(i * 16, 16)]
            dv = dst_v[pl.ds(i * 16, 16)]
            psrc = sv + jnp.where(sv >= HALF, 120, 0).astype(jnp.int32)
            psrc_b[i // per_row, pl.ds((i % per_row) * 16, 16)] = psrc
            prow = jnp.where(dv < HALF, dv,
                             jnp.where(dv < N, dv + 120, HALF + (dv & 63)))
            row_b[i // per_row, pl.ds((i % per_row) * 16, 16)] = prow

        ah = [None]
        for c in range(N_CHUNKS):
            b = c & 1
            pltpu.async_copy(h_hbm.at[psrc_b.at[c]], rows_b[b], gsem).wait()
            if ah[0] is not None:
                ah[0].wait()
            ah[0] = pltpu.async_copy(
                rows_b[b], part.at[row_b.at[c]], asem, add=True)
        ah[0].wait()


@functools.cache
def _get_sc_agg():
    mesh = plsc.VectorSubcoreMesh(core_axis_name="c", subcore_axis_name="s")
    return pl.kernel(
        _sc_agg_body,
        mesh=mesh,
        out_type=jax.ShapeDtypeStruct((32, NP, H), jnp.float32),
        scratch_types=[
            pltpu.VMEM((STAGE_E,), jnp.int32),
            pltpu.VMEM((STAGE_E,), jnp.int32),
            pltpu.VMEM((N_CHUNKS, CHUNK), jnp.int32),
            pltpu.VMEM((N_CHUNKS, CHUNK), jnp.int32),
            pltpu.VMEM((CHUNK, H), jnp.float32),
            pltpu.VMEM((CHUNK, H), jnp.float32),
            pltpu.VMEM((ZR, H), jnp.float32),
            pltpu.SemaphoreType.DMA,
            pltpu.SemaphoreType.DMA,
            pltpu.SemaphoreType.DMA,
            pltpu.SemaphoreType.DMA,
            pltpu.SemaphoreType.DMA,
            pltpu.SemaphoreType.DMA,
        ],
    )


_BR = 512
_GRID = NP // _BR


def _mm_body(x_ref, w_ref, b_ref, o_ref):
    o_ref[...] = (jnp.dot(x_ref[...], w_ref[...],
                          preferred_element_type=jnp.float32) + b_ref[...])


def _ln_relu_mm_body(ap_ref, g_ref, bt_ref, w_ref, b_ref, o_ref):
    a = jnp.sum(ap_ref[...], axis=0)
    mu = jnp.mean(a, axis=1, keepdims=True)
    var = jnp.mean((a - mu) ** 2, axis=1, keepdims=True)
    hn = (a - mu) * lax.rsqrt(var + 1e-5) * g_ref[...] + bt_ref[...]
    h = jnp.maximum(hn, 0.0)
    o_ref[...] = (jnp.dot(h, w_ref[...],
                          preferred_element_type=jnp.float32) + b_ref[...])


def _ln_relu_out_body(ap_ref, g_ref, bt_ref, w_ref, b_ref, o_ref):
    a = jnp.sum(ap_ref[...], axis=0)
    mu = jnp.mean(a, axis=1, keepdims=True)
    var = jnp.mean((a - mu) ** 2, axis=1, keepdims=True)
    hn = (a - mu) * lax.rsqrt(var + 1e-5) * g_ref[...] + bt_ref[...]
    h = jnp.maximum(hn, 0.0)
    logits = (jnp.dot(h, w_ref[...],
                      preferred_element_type=jnp.float32) + b_ref[...])
    col = lax.broadcasted_iota(jnp.int32, logits.shape, 1)
    neg = jnp.where(col < C, logits, -jnp.inf)
    m = jnp.max(neg, axis=1, keepdims=True)
    lse = m + jnp.log(jnp.sum(jnp.exp(neg - m), axis=1, keepdims=True))
    o_ref[...] = logits - lse


def _row_block(width):
    return pl.BlockSpec((_BR, width), lambda i: (i, 0))


def _const_block(shape):
    return pl.BlockSpec(shape, lambda i: (0, 0))


def _mm(x, w, b):
    fin = x.shape[1]
    return pl.pallas_call(
        _mm_body,
        grid=(_GRID,),
        in_specs=[_row_block(fin), _const_block((fin, H)), _const_block((1, H))],
        out_specs=_row_block(H),
        out_shape=jax.ShapeDtypeStruct((NP, H), jnp.float32),
    )(x, w, b.reshape(1, H))


def _part_block():
    return pl.BlockSpec((32, _BR, H), lambda i: (0, i, 0))


def _ln_relu_mm(ap, g, bt, w, b):
    return pl.pallas_call(
        _ln_relu_mm_body,
        grid=(_GRID,),
        in_specs=[_part_block(), _const_block((1, H)),
                  _const_block((1, H)), _const_block((H, H)),
                  _const_block((1, H))],
        out_specs=_row_block(H),
        out_shape=jax.ShapeDtypeStruct((NP, H), jnp.float32),
    )(ap, g.reshape(1, H), bt.reshape(1, H), w, b.reshape(1, H))


def _ln_relu_out(ap, g, bt, wp, bp):
    return pl.pallas_call(
        _ln_relu_out_body,
        grid=(_GRID,),
        in_specs=[_part_block(), _const_block((1, H)),
                  _const_block((1, H)), _const_block((H, 128)),
                  _const_block((1, 128))],
        out_specs=_row_block(128),
        out_shape=jax.ShapeDtypeStruct((NP, 128), jnp.float32),
    )(ap, g.reshape(1, H), bt.reshape(1, H), wp, bp)


def kernel(x, edge_index, W0, b0, g0, bt0, W1, b1, g1, bt1,
           W2, b2, g2, bt2, Wout, bout):
    xp = jnp.zeros((NP, F), jnp.float32)
    xp = xp.at[0:HALF].set(x[0:HALF]).at[HPAD:HPAD + HALF].set(x[HALF:N])

    dst = edge_index[0]
    src = edge_index[1]
    npad = EP - dst.shape[0]
    dstp = jnp.concatenate([dst, jnp.full((npad,), 1 << 28, jnp.int32)])
    srcp = jnp.concatenate([src, jnp.zeros((npad,), jnp.int32)])

    woutp = jnp.zeros((H, 128), jnp.float32).at[:, :C].set(Wout)
    boutp = jnp.zeros((1, 128), jnp.float32).at[0, :C].set(bout)

    sc_agg = _get_sc_agg()
    h = _mm(xp, W0, b0)
    ap = sc_agg(h, srcp, dstp)
    h = _ln_relu_mm(ap, g0, bt0, W1, b1)
    ap = sc_agg(h, srcp, dstp)
    h = _ln_relu_mm(ap, g1, bt1, W2, b2)
    ap = sc_agg(h, srcp, dstp)
    outp = _ln_relu_out(ap, g2, bt2, woutp, boutp)

    return jnp.concatenate([outp[0:HALF], outp[HPAD:HPAD + HALF]])[:, :C]


# speedup vs baseline: 11.2669x; 1.0341x over previous
import functools

import jax
import jax.numpy as jnp
from jax import lax
from jax.experimental import pallas as pl
from jax.experimental.pallas import tpu as pltpu
from jax.experimental.pallas import tpu_sc as plsc

N = 10000
F = 128
H = 256
C = 40

HALF = 5000
HPAD = 5120
NP = 2 * HPAD
ROWS_PER_TILE = NP // 32

E_TILE = 10240
EP = 32 * E_TILE
STAGE_E = 2048
N_STAGES = E_TILE // STAGE_E
CHUNK = 128
N_CHUNKS = STAGE_E // CHUNK
ZR = 128


def _sc_agg_body(h_hbm, src_hbm, dst_hbm, out_hbm,
                 dst_v, src_v, psrc_b, row_b, rows0, rows1, zbuf,
                 gsem, asem, zsem0, zsem1, zsem2, zsem3):
    cid = lax.axis_index("c")
    sid = lax.axis_index("s")
    wid = cid * 16 + sid
    tile_base = wid * E_TILE

    part = out_hbm.at[wid]

    @pl.loop(0, ZR)
    def _(r):
        @pl.loop(0, H // 16)
        def _(l):
            zbuf[r, pl.ds(l * 16, 16)] = jnp.zeros((16,), jnp.float32)

    zsems = [zsem0, zsem1, zsem2, zsem3]
    zhs = [None, None, None, None]
    for k in range(NP // ZR):
        b = k % 4
        if zhs[b] is not None:
            zhs[b].wait()
        zhs[b] = pltpu.async_copy(zbuf, part.at[pl.ds(k * ZR, ZR)], zsems[b])
    for zh in zhs:
        zh.wait()

    rows_b = [rows0, rows1]

    @pl.loop(0, N_STAGES)
    def _stage(st):
        eb = tile_base + st * STAGE_E
        pltpu.sync_copy(dst_hbm.at[pl.ds(eb, STAGE_E)], dst_v)
        pltpu.sync_copy(src_hbm.at[pl.ds(eb, STAGE_E)], src_v)

        per_row = CHUNK // 16
        for i in range(STAGE_E // 16):
            sv = src_v[pl.ds(i * 16, 16)]
            dv = dst_v[pl.ds(i * 16, 16)]
            psrc = sv + jnp.where(sv >= HALF, 120, 0).astype(jnp.int32)
            psrc_b[i // per_row, pl.ds((i % per_row) * 16, 16)] = psrc
            prow = jnp.where(dv < HALF, dv,
                             jnp.where(dv < N, dv + 120, HALF + (dv & 63)))
            row_b[i // per_row, pl.ds((i % per_row) * 16, 16)] = prow

        ah = [None]
        for c in range(N_CHUNKS):
            b = c & 1
            pltpu.async_copy(h_hbm.at[psrc_b.at[c]], rows_b[b], gsem).wait()
            if ah[0] is not None:
                ah[0].wait()
            ah[0] = pltpu.async_copy(
                rows_b[b], part.at[row_b.at[c]], asem, add=True)
        ah[0].wait()


@functools.cache
def _get_sc_agg():
    mesh = plsc.VectorSubcoreMesh(core_axis_name="c", subcore_axis_name="s")
    return pl.kernel(
        _sc_agg_body,
        mesh=mesh,
        out_type=jax.ShapeDtypeStruct((32, NP, H), jnp.float32),
        scratch_types=[
            pltpu.VMEM((STAGE_E,), jnp.int32),
            pltpu.VMEM((STAGE_E,), jnp.int32),
            pltpu.VMEM((N_CHUNKS, CHUNK), jnp.int32),
            pltpu.VMEM((N_CHUNKS, CHUNK), jnp.int32),
            pltpu.VMEM((CHUNK, H), jnp.float32),
            pltpu.VMEM((CHUNK, H), jnp.float32),
            pltpu.VMEM((ZR, H), jnp.float32),
            pltpu.SemaphoreType.DMA,
            pltpu.SemaphoreType.DMA,
            pltpu.SemaphoreType.DMA,
            pltpu.SemaphoreType.DMA,
            pltpu.SemaphoreType.DMA,
            pltpu.SemaphoreType.DMA,
        ],
    )


_BR = 512
_GRID = NP // _BR


def _mm_body(x_ref, w_ref, b_ref, o_ref):
    o_ref[...] = (jnp.dot(x_ref[...], w_ref[...],
                          preferred_element_type=jnp.float32) + b_ref[...])


def _ln_relu_mm_body(ap_ref, g_ref, bt_ref, w_ref, b_ref, o_ref):
    a = jnp.sum(ap_ref[...], axis=0)
    mu = jnp.mean(a, axis=1, keepdims=True)
    var = jnp.mean((a - mu) ** 2, axis=1, keepdims=True)
    hn = (a - mu) * lax.rsqrt(var + 1e-5) * g_ref[...] + bt_ref[...]
    h = jnp.maximum(hn, 0.0)
    o_ref[...] = (jnp.dot(h, w_ref[...],
                          preferred_element_type=jnp.float32) + b_ref[...])


def _ln_relu_out_body(ap_ref, g_ref, bt_ref, w_ref, b_ref, o_ref):
    a = jnp.sum(ap_ref[...], axis=0)
    mu = jnp.mean(a, axis=1, keepdims=True)
    var = jnp.mean((a - mu) ** 2, axis=1, keepdims=True)
    hn = (a - mu) * lax.rsqrt(var + 1e-5) * g_ref[...] + bt_ref[...]
    h = jnp.maximum(hn, 0.0)
    logits = (jnp.dot(h, w_ref[...],
                      preferred_element_type=jnp.float32) + b_ref[...])
    col = lax.broadcasted_iota(jnp.int32, logits.shape, 1)
    neg = jnp.where(col < C, logits, -jnp.inf)
    m = jnp.max(neg, axis=1, keepdims=True)
    lse = m + jnp.log(jnp.sum(jnp.exp(neg - m), axis=1, keepdims=True))
    o_ref[...] = logits - lse


def _row_block(width):
    return pl.BlockSpec((_BR, width), lambda i: (i, 0))


def _const_block(shape):
    return pl.BlockSpec(shape, lambda i: (0, 0))


def _mm(x, w, b):
    fin = x.shape[1]
    return pl.pallas_call(
        _mm_body,
        grid=(_GRID,),
        in_specs=[_row_block(fin), _const_block((fin, H)), _const_block((1, H))],
        out_specs=_row_block(H),
        out_shape=jax.ShapeDtypeStruct((NP, H), jnp.float32),
    )(x, w, b.reshape(1, H))


def _part_block():
    return pl.BlockSpec((32, _BR, H), lambda i: (0, i, 0))


def _ln_relu_mm(ap, g, bt, w, b):
    return pl.pallas_call(
        _ln_relu_mm_body,
        grid=(_GRID,),
        in_specs=[_part_block(), _const_block((1, H)),
                  _const_block((1, H)), _const_block((H, H)),
                  _const_block((1, H))],
        out_specs=_row_block(H),
        out_shape=jax.ShapeDtypeStruct((NP, H), jnp.float32),
    )(ap, g.reshape(1, H), bt.reshape(1, H), w, b.reshape(1, H))


def _ln_relu_out(ap, g, bt, wp, bp):
    return pl.pallas_call(
        _ln_relu_out_body,
        grid=(_GRID,),
        in_specs=[_part_block(), _const_block((1, H)),
                  _const_block((1, H)), _const_block((H, 128)),
                  _const_block((1, 128))],
        out_specs=_row_block(128),
        out_shape=jax.ShapeDtypeStruct((NP, 128), jnp.float32),
    )(ap, g.reshape(1, H), bt.reshape(1, H), wp, bp)


def kernel(x, edge_index, W0, b0, g0, bt0, W1, b1, g1, bt1,
           W2, b2, g2, bt2, Wout, bout):
    xp = jnp.zeros((NP, F), jnp.float32)
    xp = xp.at[0:HALF].set(x[0:HALF]).at[HPAD:HPAD + HALF].set(x[HALF:N])

    dst = edge_index[0]
    src = edge_index[1]
    npad = EP - dst.shape[0]
    dstp = jnp.concatenate([dst, jnp.full((npad,), 1 << 28, jnp.int32)])
    srcp = jnp.concatenate([src, jnp.zeros((npad,), jnp.int32)])

    woutp = jnp.zeros((H, 128), jnp.float32).at[:, :C].set(Wout)
    boutp = jnp.zeros((1, 128), jnp.float32).at[0, :C].set(bout)

    sc_agg = _get_sc_agg()
    h = _mm(xp, W0, b0)
    ap = sc_agg(h, srcp, dstp)
    h = _ln_relu_mm(ap, g0, bt0, W1, b1)
    ap = sc_agg(h, srcp, dstp)
    h = _ln_relu_mm(ap, g1, bt1, W2, b2)
    ap = sc_agg(h, srcp, dstp)
    outp = _ln_relu_out(ap, g2, bt2, woutp, boutp)

    return jnp.concatenate([outp[0:HALF], outp[HPAD:HPAD + HALF]])[:, :C]
